# initial kernel scaffold (unmeasured)
import jax
import jax.numpy as jnp
from jax import lax
from jax.experimental import pallas as pl
from jax.experimental.pallas import tpu as pltpu

N_DEV = 4
M_PER = 2048
K = 8192
N_PER = 1024
K_BLK = 1024
K_STEPS = K // K_BLK


def kernel(x, w_mat):
    def body(x_ref, w_ref, out_ref, y_buf, send_sems, recv_sems):
        j = pl.program_id(0)
        k = pl.program_id(1)
        me = lax.axis_index("i")

        @pl.when((j == 0) & (k == 0))
        def _entry_barrier():
            barrier_sem = pltpu.get_barrier_semaphore()
            for step in range(1, N_DEV):
                pl.semaphore_signal(
                    barrier_sem, inc=1,
                    device_id=((me + step) % N_DEV,),
                    device_id_type=pl.DeviceIdType.MESH,
                )
            pl.semaphore_wait(barrier_sem, N_DEV - 1)

        partial = jnp.dot(
            x_ref[...], w_ref[...], preferred_element_type=jnp.float32
        )

        @pl.when(k == 0)
        def _():
            y_buf[j] = partial

        @pl.when(k > 0)
        def _():
            y_buf[j] += partial

        my_rows = pl.ds(me * M_PER, M_PER)

        @pl.when(k == K_STEPS - 1)
        def _finish_block():
            y_buf[j] = jnp.maximum(y_buf[j], 0.0)

            @pl.when(j != me)
            def _send():
                rdma = pltpu.make_async_remote_copy(
                    src_ref=y_buf.at[j],
                    dst_ref=out_ref.at[my_rows, :],
                    send_sem=send_sems.at[j],
                    recv_sem=recv_sems.at[me],
                    device_id=(j,),
                    device_id_type=pl.DeviceIdType.MESH,
                )
                rdma.start()

            @pl.when(j == me)
            def _local():
                pltpu.make_async_copy(
                    y_buf.at[j], out_ref.at[my_rows, :], send_sems.at[j]
                ).start()

        @pl.when((j == N_DEV - 1) & (k == K_STEPS - 1))
        def _drain():
            pltpu.make_async_copy(
                y_buf.at[me], out_ref.at[my_rows, :], send_sems.at[me]
            ).wait()
            for step in range(1, N_DEV):
                s = (me + step) % N_DEV
                pltpu.make_async_remote_copy(
                    src_ref=y_buf.at[s],
                    dst_ref=out_ref.at[my_rows, :],
                    send_sem=send_sems.at[s],
                    recv_sem=recv_sems.at[me],
                    device_id=(s,),
                    device_id_type=pl.DeviceIdType.MESH,
                ).wait_send()
                pltpu.make_async_remote_copy(
                    src_ref=y_buf.at[s],
                    dst_ref=out_ref.at[pl.ds(s * M_PER, M_PER), :],
                    send_sem=send_sems.at[s],
                    recv_sem=recv_sems.at[s],
                    device_id=(s,),
                    device_id_type=pl.DeviceIdType.MESH,
                ).wait_recv()

    return pl.pallas_call(
        body,
        grid=(N_DEV, K_STEPS),
        in_specs=[
            pl.BlockSpec((M_PER, K_BLK), lambda j, k: (0, k)),
            pl.BlockSpec((K_BLK, N_PER), lambda j, k: (k, j)),
        ],
        out_specs=pl.BlockSpec(memory_space=pltpu.ANY),
        out_shape=jax.ShapeDtypeStruct((N_DEV * M_PER, N_PER), jnp.float32),
        scratch_shapes=[
            pltpu.VMEM((N_DEV, M_PER, N_PER), jnp.float32),
            pltpu.SemaphoreType.DMA((N_DEV,)),
            pltpu.SemaphoreType.DMA((N_DEV,)),
        ],
        compiler_params=pltpu.CompilerParams(
            dimension_semantics=("arbitrary", "arbitrary"),
            collective_id=0,
        ),
    )(x, w_mat)


# baseline (device time: 394103 ns/iter reference)
import jax
import jax.numpy as jnp
from jax import lax
from jax.experimental import pallas as pl
from jax.experimental.pallas import tpu as pltpu

N_DEV = 4
M_PER = 2048
K = 8192
N_PER = 1024
K_BLK = 1024
K_STEPS = K // K_BLK


def kernel(x, w_mat):
    def body(x_ref, w_ref, out_ref, y_buf, send_sems, recv_sems):
        j = pl.program_id(0)
        k = pl.program_id(1)
        me = lax.axis_index("i")

        @pl.when((j == 0) & (k == 0))
        def _entry_barrier():
            barrier_sem = pltpu.get_barrier_semaphore()
            for step in range(1, N_DEV):
                pl.semaphore_signal(
                    barrier_sem, inc=1,
                    device_id=((me + step) % N_DEV,),
                    device_id_type=pl.DeviceIdType.MESH,
                )
            pl.semaphore_wait(barrier_sem, N_DEV - 1)

        partial = jnp.dot(
            x_ref[...], w_ref[...], preferred_element_type=jnp.float32
        )

        @pl.when(k == 0)
        def _():
            y_buf[j] = partial

        @pl.when(k > 0)
        def _():
            y_buf[j] += partial

        my_rows = pl.ds(me * M_PER, M_PER)

        @pl.when(k == K_STEPS - 1)
        def _finish_block():
            y_buf[j] = jnp.maximum(y_buf[j], 0.0)

            @pl.when(j != me)
            def _send():
                rdma = pltpu.make_async_remote_copy(
                    src_ref=y_buf.at[j],
                    dst_ref=out_ref.at[my_rows, :],
                    send_sem=send_sems.at[j],
                    recv_sem=recv_sems.at[me],
                    device_id=(j,),
                    device_id_type=pl.DeviceIdType.MESH,
                )
                rdma.start()

            @pl.when(j == me)
            def _local():
                pltpu.make_async_copy(
                    y_buf.at[j], out_ref.at[my_rows, :], send_sems.at[j]
                ).start()

        @pl.when((j == N_DEV - 1) & (k == K_STEPS - 1))
        def _drain():
            pltpu.make_async_copy(
                y_buf.at[me], out_ref.at[my_rows, :], send_sems.at[me]
            ).wait()
            for step in range(1, N_DEV):
                s = (me + step) % N_DEV
                pltpu.make_async_remote_copy(
                    src_ref=y_buf.at[s],
                    dst_ref=out_ref.at[my_rows, :],
                    send_sem=send_sems.at[s],
                    recv_sem=recv_sems.at[me],
                    device_id=(s,),
                    device_id_type=pl.DeviceIdType.MESH,
                ).wait_send()
                pltpu.make_async_remote_copy(
                    src_ref=y_buf.at[s],
                    dst_ref=out_ref.at[pl.ds(s * M_PER, M_PER), :],
                    send_sem=send_sems.at[s],
                    recv_sem=recv_sems.at[s],
                    device_id=(s,),
                    device_id_type=pl.DeviceIdType.MESH,
                ).wait_recv()

    return pl.pallas_call(
        body,
        grid=(N_DEV, K_STEPS),
        in_specs=[
            pl.BlockSpec((M_PER, K_BLK), lambda j, k: (0, k)),
            pl.BlockSpec((K_BLK, N_PER), lambda j, k: (k, j)),
        ],
        out_specs=pl.BlockSpec(memory_space=pltpu.MemorySpace.HBM),
        out_shape=jax.ShapeDtypeStruct((N_DEV * M_PER, N_PER), jnp.float32),
        scratch_shapes=[
            pltpu.VMEM((N_DEV, M_PER, N_PER), jnp.float32),
            pltpu.SemaphoreType.DMA((N_DEV,)),
            pltpu.SemaphoreType.DMA((N_DEV,)),
        ],
        compiler_params=pltpu.CompilerParams(
            dimension_semantics=("arbitrary", "arbitrary"),
            collective_id=0,
            vmem_limit_bytes=100 * 1024 * 1024,
        ),
    )(x, w_mat)


# device time: 304128 ns/iter; 1.2958x vs baseline; 1.2958x over previous
import jax
import jax.numpy as jnp
from jax import lax
from jax.experimental import pallas as pl
from jax.experimental.pallas import tpu as pltpu

N_DEV = 4
M_PER = 2048
K = 8192
N_PER = 1024
K_BLK = 1024
K_STEPS = K // K_BLK


def kernel(x, w_mat):
    def body(x_ref, w_ref, out_ref, y_buf, send_sems, recv_sems):
        jj = pl.program_id(0)
        k = pl.program_id(1)
        me = lax.axis_index("i")
        j = lax.rem(me + 1 + jj, N_DEV)

        @pl.when((jj == 0) & (k == 0))
        def _entry_barrier():
            barrier_sem = pltpu.get_barrier_semaphore()
            for step in range(1, N_DEV):
                pl.semaphore_signal(
                    barrier_sem, inc=1,
                    device_id=((me + step) % N_DEV,),
                    device_id_type=pl.DeviceIdType.MESH,
                )
            pl.semaphore_wait(barrier_sem, N_DEV - 1)

        partial = jnp.dot(
            x_ref[...], w_ref[...], preferred_element_type=jnp.float32
        )

        @pl.when(k == 0)
        def _():
            y_buf[j] = partial

        @pl.when(k > 0)
        def _():
            y_buf[j] += partial

        my_rows = pl.ds(me * M_PER, M_PER)

        @pl.when(k == K_STEPS - 1)
        def _finish_block():
            y_buf[j] = jnp.maximum(y_buf[j], 0.0)

            @pl.when(jj != N_DEV - 1)
            def _send():
                rdma = pltpu.make_async_remote_copy(
                    src_ref=y_buf.at[j],
                    dst_ref=out_ref.at[my_rows, :],
                    send_sem=send_sems.at[j],
                    recv_sem=recv_sems.at[me],
                    device_id=(j,),
                    device_id_type=pl.DeviceIdType.MESH,
                )
                rdma.start()

            @pl.when(jj == N_DEV - 1)
            def _local():
                pltpu.make_async_copy(
                    y_buf.at[j], out_ref.at[my_rows, :], send_sems.at[j]
                ).start()

        @pl.when((jj == N_DEV - 1) & (k == K_STEPS - 1))
        def _drain():
            pltpu.make_async_copy(
                y_buf.at[me], out_ref.at[my_rows, :], send_sems.at[me]
            ).wait()
            for step in range(1, N_DEV):
                s = (me + step) % N_DEV
                pltpu.make_async_remote_copy(
                    src_ref=y_buf.at[s],
                    dst_ref=out_ref.at[my_rows, :],
                    send_sem=send_sems.at[s],
                    recv_sem=recv_sems.at[me],
                    device_id=(s,),
                    device_id_type=pl.DeviceIdType.MESH,
                ).wait_send()
                pltpu.make_async_remote_copy(
                    src_ref=y_buf.at[s],
                    dst_ref=out_ref.at[pl.ds(s * M_PER, M_PER), :],
                    send_sem=send_sems.at[s],
                    recv_sem=recv_sems.at[s],
                    device_id=(s,),
                    device_id_type=pl.DeviceIdType.MESH,
                ).wait_recv()

    return pl.pallas_call(
        body,
        grid=(N_DEV, K_STEPS),
        in_specs=[
            pl.BlockSpec((M_PER, K_BLK), lambda jj, k: (0, k)),
            pl.BlockSpec(
                (K_BLK, N_PER),
                lambda jj, k: (k, lax.rem(lax.axis_index("i") + 1 + jj, N_DEV)),
            ),
        ],
        out_specs=pl.BlockSpec(memory_space=pltpu.MemorySpace.HBM),
        out_shape=jax.ShapeDtypeStruct((N_DEV * M_PER, N_PER), jnp.float32),
        scratch_shapes=[
            pltpu.VMEM((N_DEV, M_PER, N_PER), jnp.float32),
            pltpu.SemaphoreType.DMA((N_DEV,)),
            pltpu.SemaphoreType.DMA((N_DEV,)),
        ],
        compiler_params=pltpu.CompilerParams(
            dimension_semantics=("arbitrary", "arbitrary"),
            collective_id=0,
            vmem_limit_bytes=100 * 1024 * 1024,
        ),
    )(x, w_mat)


# device time: 189235 ns/iter; 2.0826x vs baseline; 1.6071x over previous
import os

import jax
import jax.numpy as jnp
from jax import lax
from jax.experimental import pallas as pl
from jax.experimental.pallas import tpu as pltpu

N_DEV = 4
M_PER = 2048
K = 8192
N_PER = 1024
K_BLK = 1024
K_STEPS = K // K_BLK

_COMM = os.environ.get("DIAG_NO_COMM") != "1"


def kernel(x, w_mat):
    def body(x_ref, w_ref, out_ref, y_buf, send_sems, recv_sems):
        jj = pl.program_id(0)
        k = pl.program_id(1)
        me = lax.axis_index("i")
        j = lax.rem(me + 1 + jj, N_DEV)

        if _COMM:
            @pl.when((jj == 0) & (k == 0))
            def _entry_barrier():
                barrier_sem = pltpu.get_barrier_semaphore()
                for step in range(1, N_DEV):
                    pl.semaphore_signal(
                        barrier_sem, inc=1,
                        device_id=((me + step) % N_DEV,),
                        device_id_type=pl.DeviceIdType.MESH,
                    )
                pl.semaphore_wait(barrier_sem, N_DEV - 1)

        partial = jnp.dot(
            x_ref[...], w_ref[...], preferred_element_type=jnp.float32
        )

        @pl.when(k == 0)
        def _():
            y_buf[j] = partial

        @pl.when(k > 0)
        def _():
            y_buf[j] += partial

        my_rows = pl.ds(me * M_PER, M_PER)

        @pl.when(k == K_STEPS - 1)
        def _finish_block():
            y_buf[j] = jnp.maximum(y_buf[j], 0.0)

            if _COMM:
                @pl.when(jj != N_DEV - 1)
                def _send():
                    rdma = pltpu.make_async_remote_copy(
                        src_ref=y_buf.at[j],
                        dst_ref=out_ref.at[my_rows, :],
                        send_sem=send_sems.at[j],
                        recv_sem=recv_sems.at[me],
                        device_id=(j,),
                        device_id_type=pl.DeviceIdType.MESH,
                    )
                    rdma.start()

            @pl.when(jj == N_DEV - 1)
            def _local():
                pltpu.make_async_copy(
                    y_buf.at[j], out_ref.at[my_rows, :], send_sems.at[j]
                ).start()

        @pl.when((jj == N_DEV - 1) & (k == K_STEPS - 1))
        def _drain():
            pltpu.make_async_copy(
                y_buf.at[me], out_ref.at[my_rows, :], send_sems.at[me]
            ).wait()
            for step in range(1, N_DEV) if _COMM else []:
                s = (me + step) % N_DEV
                pltpu.make_async_remote_copy(
                    src_ref=y_buf.at[s],
                    dst_ref=out_ref.at[my_rows, :],
                    send_sem=send_sems.at[s],
                    recv_sem=recv_sems.at[me],
                    device_id=(s,),
                    device_id_type=pl.DeviceIdType.MESH,
                ).wait_send()
                pltpu.make_async_remote_copy(
                    src_ref=y_buf.at[s],
                    dst_ref=out_ref.at[pl.ds(s * M_PER, M_PER), :],
                    send_sem=send_sems.at[s],
                    recv_sem=recv_sems.at[s],
                    device_id=(s,),
                    device_id_type=pl.DeviceIdType.MESH,
                ).wait_recv()

    return pl.pallas_call(
        body,
        grid=(N_DEV, K_STEPS),
        in_specs=[
            pl.BlockSpec((M_PER, K_BLK), lambda jj, k: (0, k)),
            pl.BlockSpec(
                (K_BLK, N_PER),
                lambda jj, k: (k, lax.rem(lax.axis_index("i") + 1 + jj, N_DEV)),
            ),
        ],
        out_specs=pl.BlockSpec(memory_space=pltpu.MemorySpace.HBM),
        out_shape=jax.ShapeDtypeStruct((N_DEV * M_PER, N_PER), jnp.float32),
        scratch_shapes=[
            pltpu.VMEM((N_DEV, M_PER, N_PER), jnp.float32),
            pltpu.SemaphoreType.DMA((N_DEV,)),
            pltpu.SemaphoreType.DMA((N_DEV,)),
        ],
        compiler_params=pltpu.CompilerParams(
            dimension_semantics=("arbitrary", "arbitrary"),
            collective_id=0 if _COMM else None,
            vmem_limit_bytes=100 * 1024 * 1024,
        ),
    )(x, w_mat)
